# baseline (speedup 1.0000x reference)
import dataclasses
import functools

import jax
import jax.numpy as jnp
from jax import lax
from jax.experimental import pallas as pl
from jax.experimental.pallas import tpu as pltpu
from jax.experimental.pallas import tpu_sc as plsc

N = 10000
E = 320000
D = 128
HD = 128

ROWS = 2000

NC, NS, LANES = 2, 16, 16
NW = NC * NS
EPW = E // NW
CHUNK = 80
NCHUNK = EPW // CHUNK
GROUPS = CHUNK // LANES


def _dense_body(x_ref, h0_ref, c0_ref, wg_ref, wcg_ref, bg_ref,
                wl_ref, bl_ref, wsd_ref, bsd_ref,
                hn_ref, c_ref, a_ref, b_ref):
    x = x_ref[...]
    h0 = h0_ref[...]
    g = (jnp.dot(x, wg_ref[...], preferred_element_type=jnp.float32)
         + jnp.dot(h0, wcg_ref[...], preferred_element_type=jnp.float32)
         + bg_ref[...])
    i = jax.nn.sigmoid(g[:, 0 * HD:1 * HD])
    f = jax.nn.sigmoid(g[:, 1 * HD:2 * HD])
    t = jnp.tanh(g[:, 2 * HD:3 * HD])
    o = jax.nn.sigmoid(g[:, 3 * HD:4 * HD])
    c = f * c0_ref[...] + i * t
    hn = o * jnp.tanh(c)
    z = (jnp.dot(jax.nn.relu(hn), wl_ref[...],
                 preferred_element_type=jnp.float32) + bl_ref[...])
    ab = (jnp.dot(z, wsd_ref[...], preferred_element_type=jnp.float32)
          + bsd_ref[...])
    hn_ref[...] = hn
    c_ref[...] = c
    a_ref[...] = ab[:, :HD]
    b_ref[...] = ab[:, HD:]


def _dense_stage(node_feat, h0, c0, wg, wcg, bg, wl, bl, wsd, bsd):
    row_spec = pl.BlockSpec((ROWS, D), lambda i: (i, 0))
    full = lambda s: pl.BlockSpec(s, lambda i: (0,) * len(s))
    return pl.pallas_call(
        _dense_body,
        grid=(N // ROWS,),
        in_specs=[row_spec, row_spec, row_spec,
                  full((D, 4 * HD)), full((HD, 4 * HD)), full((1, 4 * HD)),
                  full((HD, HD)), full((1, HD)),
                  full((HD, 2 * HD)), full((1, 2 * HD))],
        out_specs=[row_spec, row_spec, row_spec, row_spec],
        out_shape=[jax.ShapeDtypeStruct((N, HD), jnp.float32)] * 4,
    )(node_feat, h0, c0, wg, wcg, bg, wl, bl, wsd, bsd)


def _edge_body(a_hbm, b_hbm, src_hbm, dst_hbm, neg_hbm, wf_hbm, bf_hbm,
               pos_hbm, negout_hbm,
               idx_s, idx_d, idx_n, rows_a, rows_b, rows_n,
               wf_v, bf_v, pos_v, neg_v, sem):
    wid = lax.axis_index("s") * NC + lax.axis_index("c")
    base = wid * EPW
    pltpu.sync_copy(wf_hbm, wf_v)
    pltpu.sync_copy(bf_hbm, bf_v)
    lane = lax.iota(jnp.int32, 16)
    eidx = [lane + g * LANES for g in range(GROUPS)]

    @pl.loop(0, NCHUNK)
    def _(cc):
        cbase = base + cc * CHUNK
        pltpu.sync_copy(src_hbm.at[pl.ds(cbase, CHUNK)], idx_s)
        pltpu.sync_copy(dst_hbm.at[pl.ds(cbase, CHUNK)], idx_d)
        pltpu.sync_copy(neg_hbm.at[pl.ds(cbase, CHUNK)], idx_n)
        ca = pltpu.async_copy(a_hbm.at[idx_s], rows_a, sem)
        cb = pltpu.async_copy(b_hbm.at[idx_d], rows_b, sem)
        cn = pltpu.async_copy(b_hbm.at[idx_n], rows_n, sem)
        ca.wait()
        cb.wait()
        cn.wait()
        bf = bf_v[...]
        for g in range(GROUPS):
            def jstep(j, carry):
                acc_p, acc_n = carry
                colj = jnp.full((16,), j, dtype=jnp.int32)
                va = plsc.load_gather(rows_a, [eidx[g], colj])
                vb = plsc.load_gather(rows_b, [eidx[g], colj])
                vn = plsc.load_gather(rows_n, [eidx[g], colj])
                wb = wf_v[j, :]
                up = jnp.maximum(va + vb, 0.0)
                un = jnp.maximum(va + vn, 0.0)
                return acc_p + up * wb, acc_n + un * wb
            acc_p, acc_n = lax.fori_loop(0, HD, jstep, (bf, bf))
            pos_v[pl.ds(g * LANES, LANES)] = 1.0 / (1.0 + jnp.exp(-acc_p))
            neg_v[pl.ds(g * LANES, LANES)] = 1.0 / (1.0 + jnp.exp(-acc_n))
        pltpu.sync_copy(pos_v, pos_hbm.at[pl.ds(cbase, CHUNK)])
        pltpu.sync_copy(neg_v, negout_hbm.at[pl.ds(cbase, CHUNK)])


def _edge_stage(a, b, src, dst, neg, wf_bcast, bf_bcast):
    mesh = plsc.VectorSubcoreMesh(core_axis_name="c", subcore_axis_name="s")
    cp = pltpu.CompilerParams()
    if "needs_layout_passes" in pltpu.CompilerParams.__dataclass_fields__:
        cp = dataclasses.replace(cp, needs_layout_passes=False)
    k = pl.kernel(
        _edge_body,
        out_type=(jax.ShapeDtypeStruct((E,), jnp.float32),
                  jax.ShapeDtypeStruct((E,), jnp.float32)),
        mesh=mesh,
        scratch_types=[
            pltpu.VMEM((CHUNK,), jnp.int32),
            pltpu.VMEM((CHUNK,), jnp.int32),
            pltpu.VMEM((CHUNK,), jnp.int32),
            pltpu.VMEM((CHUNK, HD), jnp.float32),
            pltpu.VMEM((CHUNK, HD), jnp.float32),
            pltpu.VMEM((CHUNK, HD), jnp.float32),
            pltpu.VMEM((HD, 16), jnp.float32),
            pltpu.VMEM((16,), jnp.float32),
            pltpu.VMEM((CHUNK,), jnp.float32),
            pltpu.VMEM((CHUNK,), jnp.float32),
            pltpu.SemaphoreType.DMA,
        ],
        compiler_params=cp,
    )
    return k(a, b, src, dst, neg, wf_bcast, bf_bcast)


def kernel(node_feat, src, dst, neg, h0, c0,
           W_i, b_i, Wc_i, bc_i, W_f, b_f, Wc_f, bc_f,
           W_c, b_c, Wc_c, bc_c, W_o, b_o, Wc_o, bc_o,
           Wl, bl, Ws, bs, Wd, bd, Wfin, bfin):
    wg = jnp.concatenate([W_i, W_f, W_c, W_o], axis=1)
    wcg = jnp.concatenate([Wc_i, Wc_f, Wc_c, Wc_o], axis=1)
    bg = jnp.concatenate([b_i[0] + bc_i, b_f[0] + bc_f,
                          b_c[0] + bc_c, b_o[0] + bc_o]).reshape(1, 4 * HD)
    wsd = jnp.concatenate([Ws, Wd], axis=1)
    bsd = jnp.concatenate([bs, bd]).reshape(1, 2 * HD)

    hn, c, a, b = _dense_stage(node_feat, h0, c0, wg, wcg, bg,
                               Wl, bl.reshape(1, HD), wsd, bsd)

    wf_bcast = jnp.broadcast_to(Wfin, (HD, 16))
    bf_bcast = jnp.broadcast_to(bfin, (16,))
    pos, negv = _edge_stage(a, b, src, dst, neg, wf_bcast, bf_bcast)
    return (pos.reshape(E, 1), negv.reshape(E, 1), hn, c)


# speedup vs baseline: 5.5332x; 5.5332x over previous
import dataclasses
import functools

import jax
import jax.numpy as jnp
from jax import lax
from jax.experimental import pallas as pl
from jax.experimental.pallas import tpu as pltpu
from jax.experimental.pallas import tpu_sc as plsc

N = 10000
E = 320000
D = 128
HD = 128

ROWS = 2000

NC, NS, LANES = 2, 16, 16
NW = NC * NS
EPW = E // NW
CHUNK = 80
NCHUNK = EPW // CHUNK
GROUPS = CHUNK // LANES
ROWS3 = 3 * CHUNK
NJ = HD // LANES


def _dense_body(x_ref, h0_ref, c0_ref, wg_ref, wcg_ref, bg_ref,
                wl_ref, bl_ref, wsd_ref, bsd_ref,
                hn_ref, c_ref, ab_ref):
    x = x_ref[...]
    h0 = h0_ref[...]
    g = (jnp.dot(x, wg_ref[...], preferred_element_type=jnp.float32)
         + jnp.dot(h0, wcg_ref[...], preferred_element_type=jnp.float32)
         + bg_ref[...])
    i = jax.nn.sigmoid(g[:, 0 * HD:1 * HD])
    f = jax.nn.sigmoid(g[:, 1 * HD:2 * HD])
    t = jnp.tanh(g[:, 2 * HD:3 * HD])
    o = jax.nn.sigmoid(g[:, 3 * HD:4 * HD])
    c = f * c0_ref[...] + i * t
    hn = o * jnp.tanh(c)
    z = (jnp.dot(jax.nn.relu(hn), wl_ref[...],
                 preferred_element_type=jnp.float32) + bl_ref[...])
    ab = (jnp.dot(z, wsd_ref[...], preferred_element_type=jnp.float32)
          + bsd_ref[...])
    hn_ref[...] = hn
    c_ref[...] = c
    ab_ref[0] = ab[:, :HD]
    ab_ref[1] = ab[:, HD:]


def _dense_stage(node_feat, h0, c0, wg, wcg, bg, wl, bl, wsd, bsd):
    row_spec = pl.BlockSpec((ROWS, D), lambda i: (i, 0))
    full = lambda s: pl.BlockSpec(s, lambda i: (0,) * len(s))
    return pl.pallas_call(
        _dense_body,
        grid=(N // ROWS,),
        in_specs=[row_spec, row_spec, row_spec,
                  full((D, 4 * HD)), full((HD, 4 * HD)), full((1, 4 * HD)),
                  full((HD, HD)), full((1, HD)),
                  full((HD, 2 * HD)), full((1, 2 * HD))],
        out_specs=[row_spec, row_spec,
                   pl.BlockSpec((2, ROWS, HD), lambda i: (0, i, 0))],
        out_shape=[jax.ShapeDtypeStruct((N, HD), jnp.float32),
                   jax.ShapeDtypeStruct((N, HD), jnp.float32),
                   jax.ShapeDtypeStruct((2, N, HD), jnp.float32)],
    )(node_feat, h0, c0, wg, wcg, bg, wl, bl, wsd, bsd)


def _edge_body(tab_hbm, idx_hbm, wf_hbm, bf_hbm,
               pos_hbm, neg_hbm,
               i0, i1, r0, r1, wf_v, bf_v, pos_all, neg_all,
               sem_i0, sem_i1, sem_r0, sem_r1):
    wid = lax.axis_index("s") * NC + lax.axis_index("c")
    q0 = wid * NCHUNK
    obase_hbm = wid * EPW

    pltpu.sync_copy(wf_hbm, wf_v)
    pltpu.sync_copy(bf_hbm, bf_v)
    wfs = [wf_v[pl.ds(j * LANES, LANES)] for j in range(NJ)]
    bfv = bf_v[...]
    zero = jnp.zeros((LANES,), jnp.float32)
    last_lane = lax.iota(jnp.int32, LANES) == (LANES - 1)

    def copy_idx(q, iref, sem):
        pltpu.async_copy(idx_hbm.at[pl.ds(q * ROWS3, ROWS3)], iref, sem)

    def wait_idx(iref, sem):
        pltpu.make_async_copy(idx_hbm.at[pl.ds(0, ROWS3)], iref, sem).wait()

    def gather(iref, rref, sem):
        pltpu.async_copy(tab_hbm.at[iref], rref, sem)

    def wait_rows(iref, rref, sem):
        pltpu.make_async_copy(tab_hbm.at[iref], rref, sem).wait()

    def compute(rref, c):
        obase = c * CHUNK

        @pl.loop(0, CHUNK, step=2)
        def _(e):
            for de in range(2):
                ee = e + de
                accs = [bfv, zero, bfv, zero]
                for j in range(NJ):
                    sl = pl.ds(j * LANES, LANES)
                    va = rref[ee, sl]
                    vb = rref[CHUNK + ee, sl]
                    vn = rref[2 * CHUNK + ee, sl]
                    up = jnp.maximum(va + vb, 0.0)
                    un = jnp.maximum(va + vn, 0.0)
                    k = j % 2
                    accs[k] = accs[k] + up * wfs[j]
                    accs[2 + k] = accs[2 + k] + un * wfs[j]
                tgt = jnp.full((LANES,), obase + ee, jnp.int32)
                plsc.store_scatter(pos_all, [tgt],
                                   plsc.cumsum(accs[0] + accs[1]),
                                   mask=last_lane)
                plsc.store_scatter(neg_all, [tgt],
                                   plsc.cumsum(accs[2] + accs[3]),
                                   mask=last_lane)

        for g in range(GROUPS):
            sl = pl.ds(obase + g * LANES, LANES)
            vp = pos_all[sl]
            vq = neg_all[sl]
            pos_all[sl] = 1.0 / (1.0 + jnp.exp(-vp))
            neg_all[sl] = 1.0 / (1.0 + jnp.exp(-vq))

    copy_idx(q0, i0, sem_i0)
    wait_idx(i0, sem_i0)
    gather(i0, r0, sem_r0)
    copy_idx(q0 + 1, i1, sem_i1)

    @pl.loop(0, NCHUNK - 1, step=2)
    def _(c):
        wait_idx(i1, sem_i1)
        gather(i1, r1, sem_r1)
        wait_rows(i0, r0, sem_r0)
        copy_idx(q0 + c + 2, i0, sem_i0)
        compute(r0, c)
        wait_idx(i0, sem_i0)
        gather(i0, r0, sem_r0)
        copy_idx(q0 + jnp.minimum(c + 3, NCHUNK - 1), i1, sem_i1)
        wait_rows(i1, r1, sem_r1)
        compute(r1, c + 1)

    wait_idx(i1, sem_i1)
    wait_rows(i0, r0, sem_r0)
    compute(r0, NCHUNK - 1)
    pltpu.sync_copy(pos_all, pos_hbm.at[pl.ds(obase_hbm, EPW)])
    pltpu.sync_copy(neg_all, neg_hbm.at[pl.ds(obase_hbm, EPW)])


def _edge_stage(tab, idx_all, wf, bf):
    mesh = plsc.VectorSubcoreMesh(core_axis_name="c", subcore_axis_name="s")
    cp = pltpu.CompilerParams()
    if "needs_layout_passes" in pltpu.CompilerParams.__dataclass_fields__:
        cp = dataclasses.replace(cp, needs_layout_passes=False)
    k = pl.kernel(
        _edge_body,
        out_type=(jax.ShapeDtypeStruct((E,), jnp.float32),
                  jax.ShapeDtypeStruct((E,), jnp.float32)),
        mesh=mesh,
        scratch_types=[
            pltpu.VMEM((ROWS3,), jnp.int32),
            pltpu.VMEM((ROWS3,), jnp.int32),
            pltpu.VMEM((ROWS3, HD), jnp.float32),
            pltpu.VMEM((ROWS3, HD), jnp.float32),
            pltpu.VMEM((HD,), jnp.float32),
            pltpu.VMEM((LANES,), jnp.float32),
            pltpu.VMEM((EPW,), jnp.float32),
            pltpu.VMEM((EPW,), jnp.float32),
            pltpu.SemaphoreType.DMA,
            pltpu.SemaphoreType.DMA,
            pltpu.SemaphoreType.DMA,
            pltpu.SemaphoreType.DMA,
        ],
        compiler_params=cp,
    )
    return k(tab, idx_all, wf, bf)


def kernel(node_feat, src, dst, neg, h0, c0,
           W_i, b_i, Wc_i, bc_i, W_f, b_f, Wc_f, bc_f,
           W_c, b_c, Wc_c, bc_c, W_o, b_o, Wc_o, bc_o,
           Wl, bl, Ws, bs, Wd, bd, Wfin, bfin):
    wg = jnp.concatenate([W_i, W_f, W_c, W_o], axis=1)
    wcg = jnp.concatenate([Wc_i, Wc_f, Wc_c, Wc_o], axis=1)
    bg = jnp.concatenate([b_i[0] + bc_i, b_f[0] + bc_f,
                          b_c[0] + bc_c, b_o[0] + bc_o]).reshape(1, 4 * HD)
    wsd = jnp.concatenate([Ws, Wd], axis=1)
    bsd = jnp.concatenate([bs, bd]).reshape(1, 2 * HD)

    hn, c, ab = _dense_stage(node_feat, h0, c0, wg, wcg, bg,
                             Wl, bl.reshape(1, HD), wsd, bsd)
    tab = ab.reshape(2 * N, HD)

    idx_all = (jnp.stack([src, dst + N, neg + N])
               .reshape(3, NW * NCHUNK, CHUNK)
               .transpose(1, 0, 2)
               .reshape(-1))

    wf = Wfin.reshape(HD)
    bf = jnp.full((LANES,), bfin[0] / LANES, dtype=jnp.float32)
    pos, negv = _edge_stage(tab, idx_all, wf, bf)
    return (pos.reshape(E, 1), negv.reshape(E, 1), hn, c)


# speedup vs baseline: 5.5451x; 1.0021x over previous
import dataclasses
import functools

import jax
import jax.numpy as jnp
from jax import lax
from jax.experimental import pallas as pl
from jax.experimental.pallas import tpu as pltpu
from jax.experimental.pallas import tpu_sc as plsc

N = 10000
E = 320000
D = 128
HD = 128

ROWS = 2000

NC, NS, LANES = 2, 16, 16
NW = NC * NS
EPW = E // NW
CHUNK = 80
NCHUNK = EPW // CHUNK
GROUPS = CHUNK // LANES
ROWS3 = 3 * CHUNK
NJ = HD // LANES


def _dense_body(x_ref, h0_ref, c0_ref, wg_ref, wcg_ref, bg_ref,
                wl_ref, bl_ref, wsd_ref, bsd_ref,
                hn_ref, c_ref, ab_ref):
    x = x_ref[...]
    h0 = h0_ref[...]
    g = (jnp.dot(x, wg_ref[...], preferred_element_type=jnp.float32)
         + jnp.dot(h0, wcg_ref[...], preferred_element_type=jnp.float32)
         + bg_ref[...])
    i = jax.nn.sigmoid(g[:, 0 * HD:1 * HD])
    f = jax.nn.sigmoid(g[:, 1 * HD:2 * HD])
    t = jnp.tanh(g[:, 2 * HD:3 * HD])
    o = jax.nn.sigmoid(g[:, 3 * HD:4 * HD])
    c = f * c0_ref[...] + i * t
    hn = o * jnp.tanh(c)
    z = (jnp.dot(jax.nn.relu(hn), wl_ref[...],
                 preferred_element_type=jnp.float32) + bl_ref[...])
    ab = (jnp.dot(z, wsd_ref[...], preferred_element_type=jnp.float32)
          + bsd_ref[...])
    hn_ref[...] = hn
    c_ref[...] = c
    ab_ref[0] = ab[:, :HD]
    ab_ref[1] = ab[:, HD:]


def _dense_stage(node_feat, h0, c0, wg, wcg, bg, wl, bl, wsd, bsd):
    row_spec = pl.BlockSpec((ROWS, D), lambda i: (i, 0))
    full = lambda s: pl.BlockSpec(s, lambda i: (0,) * len(s))
    return pl.pallas_call(
        _dense_body,
        grid=(N // ROWS,),
        in_specs=[row_spec, row_spec, row_spec,
                  full((D, 4 * HD)), full((HD, 4 * HD)), full((1, 4 * HD)),
                  full((HD, HD)), full((1, HD)),
                  full((HD, 2 * HD)), full((1, 2 * HD))],
        out_specs=[row_spec, row_spec,
                   pl.BlockSpec((2, ROWS, HD), lambda i: (0, i, 0))],
        out_shape=[jax.ShapeDtypeStruct((N, HD), jnp.float32),
                   jax.ShapeDtypeStruct((N, HD), jnp.float32),
                   jax.ShapeDtypeStruct((2, N, HD), jnp.float32)],
    )(node_feat, h0, c0, wg, wcg, bg, wl, bl, wsd, bsd)


def _edge_body(tab_hbm, idx_hbm, wf_hbm, bf_hbm,
               pos_hbm, neg_hbm,
               i0, i1, r0, r1, wf_v, bf_v, pos_all, neg_all,
               sem_i0, sem_i1, sem_r0, sem_r1):
    wid = lax.axis_index("s") * NC + lax.axis_index("c")
    q0 = wid * NCHUNK
    obase_hbm = wid * EPW

    pltpu.sync_copy(wf_hbm, wf_v)
    pltpu.sync_copy(bf_hbm, bf_v)
    wfs = [wf_v[pl.ds(j * LANES, LANES)] for j in range(NJ)]
    bfv = bf_v[...]
    zero = jnp.zeros((LANES,), jnp.float32)
    last_lane = lax.iota(jnp.int32, LANES) == (LANES - 1)

    def copy_idx(q, iref, sem):
        pltpu.async_copy(idx_hbm.at[pl.ds(q * ROWS3, ROWS3)], iref, sem)

    def wait_idx(iref, sem):
        pltpu.make_async_copy(idx_hbm.at[pl.ds(0, ROWS3)], iref, sem).wait()

    def gather(iref, rref, sem):
        pltpu.async_copy(tab_hbm.at[iref], rref, sem)

    def wait_rows(iref, rref, sem):
        pltpu.make_async_copy(tab_hbm.at[iref], rref, sem).wait()

    def compute(rref, c):
        obase = c * CHUNK

        @pl.loop(0, CHUNK, step=4)
        def _(e):
            for de in range(4):
                ee = e + de
                accs = [bfv, zero, bfv, zero]
                for j in range(NJ):
                    sl = pl.ds(j * LANES, LANES)
                    va = rref[ee, sl]
                    vb = rref[CHUNK + ee, sl]
                    vn = rref[2 * CHUNK + ee, sl]
                    up = jnp.maximum(va + vb, 0.0)
                    un = jnp.maximum(va + vn, 0.0)
                    k = j % 2
                    accs[k] = accs[k] + up * wfs[j]
                    accs[2 + k] = accs[2 + k] + un * wfs[j]
                tgt = jnp.full((LANES,), obase + ee, jnp.int32)
                plsc.store_scatter(pos_all, [tgt],
                                   plsc.cumsum(accs[0] + accs[1]),
                                   mask=last_lane)
                plsc.store_scatter(neg_all, [tgt],
                                   plsc.cumsum(accs[2] + accs[3]),
                                   mask=last_lane)

        for g in range(GROUPS):
            sl = pl.ds(obase + g * LANES, LANES)
            vp = pos_all[sl]
            vq = neg_all[sl]
            pos_all[sl] = 1.0 / (1.0 + jnp.exp(-vp))
            neg_all[sl] = 1.0 / (1.0 + jnp.exp(-vq))

    copy_idx(q0, i0, sem_i0)
    wait_idx(i0, sem_i0)
    gather(i0, r0, sem_r0)
    copy_idx(q0 + 1, i1, sem_i1)

    @pl.loop(0, NCHUNK - 1, step=2)
    def _(c):
        wait_idx(i1, sem_i1)
        gather(i1, r1, sem_r1)
        wait_rows(i0, r0, sem_r0)
        copy_idx(q0 + c + 2, i0, sem_i0)
        compute(r0, c)
        wait_idx(i0, sem_i0)
        gather(i0, r0, sem_r0)
        copy_idx(q0 + jnp.minimum(c + 3, NCHUNK - 1), i1, sem_i1)
        wait_rows(i1, r1, sem_r1)
        compute(r1, c + 1)

    wait_idx(i1, sem_i1)
    wait_rows(i0, r0, sem_r0)
    compute(r0, NCHUNK - 1)
    pltpu.sync_copy(pos_all, pos_hbm.at[pl.ds(obase_hbm, EPW)])
    pltpu.sync_copy(neg_all, neg_hbm.at[pl.ds(obase_hbm, EPW)])


def _edge_stage(tab, idx_all, wf, bf):
    mesh = plsc.VectorSubcoreMesh(core_axis_name="c", subcore_axis_name="s")
    cp = pltpu.CompilerParams()
    if "needs_layout_passes" in pltpu.CompilerParams.__dataclass_fields__:
        cp = dataclasses.replace(cp, needs_layout_passes=False)
    k = pl.kernel(
        _edge_body,
        out_type=(jax.ShapeDtypeStruct((E,), jnp.float32),
                  jax.ShapeDtypeStruct((E,), jnp.float32)),
        mesh=mesh,
        scratch_types=[
            pltpu.VMEM((ROWS3,), jnp.int32),
            pltpu.VMEM((ROWS3,), jnp.int32),
            pltpu.VMEM((ROWS3, HD), jnp.float32),
            pltpu.VMEM((ROWS3, HD), jnp.float32),
            pltpu.VMEM((HD,), jnp.float32),
            pltpu.VMEM((LANES,), jnp.float32),
            pltpu.VMEM((EPW,), jnp.float32),
            pltpu.VMEM((EPW,), jnp.float32),
            pltpu.SemaphoreType.DMA,
            pltpu.SemaphoreType.DMA,
            pltpu.SemaphoreType.DMA,
            pltpu.SemaphoreType.DMA,
        ],
        compiler_params=cp,
    )
    return k(tab, idx_all, wf, bf)


def kernel(node_feat, src, dst, neg, h0, c0,
           W_i, b_i, Wc_i, bc_i, W_f, b_f, Wc_f, bc_f,
           W_c, b_c, Wc_c, bc_c, W_o, b_o, Wc_o, bc_o,
           Wl, bl, Ws, bs, Wd, bd, Wfin, bfin):
    wg = jnp.concatenate([W_i, W_f, W_c, W_o], axis=1)
    wcg = jnp.concatenate([Wc_i, Wc_f, Wc_c, Wc_o], axis=1)
    bg = jnp.concatenate([b_i[0] + bc_i, b_f[0] + bc_f,
                          b_c[0] + bc_c, b_o[0] + bc_o]).reshape(1, 4 * HD)
    wsd = jnp.concatenate([Ws, Wd], axis=1)
    bsd = jnp.concatenate([bs, bd]).reshape(1, 2 * HD)

    hn, c, ab = _dense_stage(node_feat, h0, c0, wg, wcg, bg,
                             Wl, bl.reshape(1, HD), wsd, bsd)
    tab = ab.reshape(2 * N, HD)

    idx_all = (jnp.stack([src, dst + N, neg + N])
               .reshape(3, NW * NCHUNK, CHUNK)
               .transpose(1, 0, 2)
               .reshape(-1))

    wf = Wfin.reshape(HD)
    bf = jnp.full((LANES,), bfin[0] / LANES, dtype=jnp.float32)
    pos, negv = _edge_stage(tab, idx_all, wf, bf)
    return (pos.reshape(E, 1), negv.reshape(E, 1), hn, c)
